# baseline (speedup 1.0000x reference)
import functools
import math

import jax
import jax.numpy as jnp
from jax.experimental import pallas as pl
from jax.experimental.pallas import tpu as pltpu

_TEMP = 0.05
_INV_TEMP = 1.0 / _TEMP
_A = _INV_TEMP * math.log2(math.e)
_B = 256
_D = 256
_N = 100000
_ROWS = 10000
_STEPS = _N // _ROWS
_GATHER_STEPS = 8
_PER_STEP = _B // _GATHER_STEPS
_DRAIN_LAG = 2


def _fused_loss_kernel(tgt_ref, ir_ref, inr_ref, frgb_ref, fnir_ref,
                       frgb_any, fnir_any,
                       out_rgb_ref, out_nir_ref,
                       irn, inrn, irs, inrs, se_rgb, se_nir, g_rgb, g_nir,
                       sem):
    i = pl.program_id(0)

    @pl.when(i == 0)
    def _init():
        for src, dst, dsts in ((ir_ref, irn, irs), (inr_ref, inrn, inrs)):
            x = src[...]
            norm = jnp.sqrt(jnp.sum(x * x, axis=1, keepdims=True))
            xn = x / jnp.maximum(norm, 1e-12)
            dst[...] = xn
            dsts[...] = (xn * _A).astype(jnp.bfloat16)
        zero = jnp.zeros((_B, 1), jnp.float32)
        se_rgb[...] = zero
        se_nir[...] = zero

    for q, f_ref, se in ((irs, frgb_ref, se_rgb), (inrs, fnir_ref, se_nir)):
        d = jax.lax.dot_general(
            q[...], f_ref[...], (((1,), (1,)), ((), ())),
            precision=jax.lax.Precision.DEFAULT,
            preferred_element_type=jnp.float32)
        e = jnp.exp2(d.astype(jnp.bfloat16)).astype(jnp.float32)
        se[...] += jnp.sum(e, axis=1, keepdims=True)

    @pl.when(i < _GATHER_STEPS)
    def _issue():
        def issue(j, _):
            t = tgt_ref[j]
            pltpu.make_async_copy(frgb_any.at[pl.ds(t, 1), :],
                                  g_rgb.at[pl.ds(j, 1), :], sem).start()
            pltpu.make_async_copy(fnir_any.at[pl.ds(t, 1), :],
                                  g_nir.at[pl.ds(j, 1), :], sem).start()
            return 0
        jax.lax.fori_loop(i * _PER_STEP, (i + 1) * _PER_STEP, issue, 0)

    @pl.when(jnp.logical_and(i >= _DRAIN_LAG, i < _DRAIN_LAG + _GATHER_STEPS))
    def _drain():
        def drain(j, _):
            pltpu.make_async_copy(frgb_any.at[pl.ds(0, 1), :],
                                  g_rgb.at[pl.ds(0, 1), :], sem).wait()
            pltpu.make_async_copy(fnir_any.at[pl.ds(0, 1), :],
                                  g_nir.at[pl.ds(0, 1), :], sem).wait()
            return 0
        jax.lax.fori_loop(0, _PER_STEP, drain, 0)

    @pl.when(i == _STEPS - 1)
    def _finish():
        for q, g, se, out in ((irn, g_rgb, se_rgb, out_rgb_ref),
                              (inrn, g_nir, se_nir, out_nir_ref)):
            tl = _INV_TEMP * jnp.sum(q[...] * g[...], axis=1, keepdims=True)
            lse = jnp.log(se[...])
            out[...] = jnp.mean(lse - tl).reshape(1, 1)


@functools.partial(jax.jit, static_argnames=())
def kernel(inputs_rgb, inputs_nir, targets, features_rgb, features_nir):
    full = lambda shape: pl.BlockSpec(shape, lambda i: (0, 0))
    out_rgb, out_nir = pl.pallas_call(
        _fused_loss_kernel,
        grid=(_STEPS,),
        in_specs=[
            pl.BlockSpec(memory_space=pltpu.MemorySpace.SMEM),
            full((_B, _D)),
            full((_B, _D)),
            pl.BlockSpec((_ROWS, _D), lambda i: (i, 0)),
            pl.BlockSpec((_ROWS, _D), lambda i: (i, 0)),
            pl.BlockSpec(memory_space=pl.ANY),
            pl.BlockSpec(memory_space=pl.ANY),
        ],
        out_specs=[full((1, 1)), full((1, 1))],
        out_shape=[jax.ShapeDtypeStruct((1, 1), jnp.float32),
                   jax.ShapeDtypeStruct((1, 1), jnp.float32)],
        scratch_shapes=[
            pltpu.VMEM((_B, _D), jnp.float32),
            pltpu.VMEM((_B, _D), jnp.float32),
            pltpu.VMEM((_B, _D), jnp.bfloat16),
            pltpu.VMEM((_B, _D), jnp.bfloat16),
            pltpu.VMEM((_B, 1), jnp.float32),
            pltpu.VMEM((_B, 1), jnp.float32),
            pltpu.VMEM((_B, _D), jnp.float32),
            pltpu.VMEM((_B, _D), jnp.float32),
            pltpu.SemaphoreType.DMA,
        ],
        compiler_params=pltpu.CompilerParams(
            dimension_semantics=("arbitrary",)),
    )(targets, inputs_rgb, inputs_nir, features_rgb, features_nir,
      features_rgb, features_nir)
    return (out_rgb.reshape(()), out_nir.reshape(()))


# speedup vs baseline: 1.0296x; 1.0296x over previous
import functools
import math

import jax
import jax.numpy as jnp
from jax.experimental import pallas as pl
from jax.experimental.pallas import tpu as pltpu

_TEMP = 0.05
_INV_TEMP = 1.0 / _TEMP
_A = _INV_TEMP * math.log2(math.e)
_B = 256
_D = 256
_N = 100000
_ROWS = 5000
_STEPS = _N // _ROWS
_GATHER_STEPS = 16
_PER_STEP = _B // _GATHER_STEPS
_DRAIN_LAG = 3


def _fused_loss_kernel(tgt_ref, ir_ref, inr_ref, frgb_ref, fnir_ref,
                       frgb_any, fnir_any,
                       out_rgb_ref, out_nir_ref,
                       irn, inrn, irs, inrs, se_rgb, se_nir, g_rgb, g_nir,
                       sem):
    i = pl.program_id(0)

    @pl.when(i == 0)
    def _init():
        for src, dst, dsts in ((ir_ref, irn, irs), (inr_ref, inrn, inrs)):
            x = src[...]
            norm = jnp.sqrt(jnp.sum(x * x, axis=1, keepdims=True))
            xn = x / jnp.maximum(norm, 1e-12)
            dst[...] = xn
            dsts[...] = xn * _A
        zero = jnp.zeros((1, _B), jnp.float32)
        se_rgb[...] = zero
        se_nir[...] = zero

    for q, f_ref, se in ((irs, frgb_ref, se_rgb), (inrs, fnir_ref, se_nir)):
        d = jax.lax.dot_general(
            f_ref[...], q[...], (((1,), (1,)), ((), ())),
            precision=jax.lax.Precision.DEFAULT,
            preferred_element_type=jnp.float32)
        e = jnp.exp2(d.astype(jnp.bfloat16)).astype(jnp.float32)
        se[...] += jnp.sum(e, axis=0, keepdims=True)

    @pl.when(i < _GATHER_STEPS)
    def _issue():
        def issue(j, _):
            t = tgt_ref[j]
            pltpu.make_async_copy(frgb_any.at[pl.ds(t, 1), :],
                                  g_rgb.at[pl.ds(j, 1), :], sem).start()
            pltpu.make_async_copy(fnir_any.at[pl.ds(t, 1), :],
                                  g_nir.at[pl.ds(j, 1), :], sem).start()
            return 0
        jax.lax.fori_loop(i * _PER_STEP, (i + 1) * _PER_STEP, issue, 0)

    @pl.when(jnp.logical_and(i >= _DRAIN_LAG, i < _DRAIN_LAG + _GATHER_STEPS))
    def _drain():
        def drain(j, _):
            pltpu.make_async_copy(frgb_any.at[pl.ds(0, 1), :],
                                  g_rgb.at[pl.ds(0, 1), :], sem).wait()
            pltpu.make_async_copy(fnir_any.at[pl.ds(0, 1), :],
                                  g_nir.at[pl.ds(0, 1), :], sem).wait()
            return 0
        jax.lax.fori_loop(0, _PER_STEP, drain, 0)

    @pl.when(i == _STEPS - 1)
    def _finish():
        for q, g, se, out in ((irn, g_rgb, se_rgb, out_rgb_ref),
                              (inrn, g_nir, se_nir, out_nir_ref)):
            tl_total = _INV_TEMP * jnp.sum(q[...] * g[...])
            lse_total = jnp.sum(jnp.log(se[...]))
            out[...] = ((lse_total - tl_total) / _B).reshape(1, 1)


@functools.partial(jax.jit, static_argnames=())
def kernel(inputs_rgb, inputs_nir, targets, features_rgb, features_nir):
    full = lambda shape: pl.BlockSpec(shape, lambda i: (0, 0))
    out_rgb, out_nir = pl.pallas_call(
        _fused_loss_kernel,
        grid=(_STEPS,),
        in_specs=[
            pl.BlockSpec(memory_space=pltpu.MemorySpace.SMEM),
            full((_B, _D)),
            full((_B, _D)),
            pl.BlockSpec((_ROWS, _D), lambda i: (i, 0)),
            pl.BlockSpec((_ROWS, _D), lambda i: (i, 0)),
            pl.BlockSpec(memory_space=pl.ANY),
            pl.BlockSpec(memory_space=pl.ANY),
        ],
        out_specs=[full((1, 1)), full((1, 1))],
        out_shape=[jax.ShapeDtypeStruct((1, 1), jnp.float32),
                   jax.ShapeDtypeStruct((1, 1), jnp.float32)],
        scratch_shapes=[
            pltpu.VMEM((_B, _D), jnp.float32),
            pltpu.VMEM((_B, _D), jnp.float32),
            pltpu.VMEM((_B, _D), jnp.float32),
            pltpu.VMEM((_B, _D), jnp.float32),
            pltpu.VMEM((1, _B), jnp.float32),
            pltpu.VMEM((1, _B), jnp.float32),
            pltpu.VMEM((_B, _D), jnp.float32),
            pltpu.VMEM((_B, _D), jnp.float32),
            pltpu.SemaphoreType.DMA,
        ],
        compiler_params=pltpu.CompilerParams(
            dimension_semantics=("arbitrary",)),
    )(targets, inputs_rgb, inputs_nir, features_rgb, features_nir,
      features_rgb, features_nir)
    return (out_rgb.reshape(()), out_nir.reshape(()))
